# baseline (speedup 1.0000x reference)
import functools

import jax
import jax.numpy as jnp
from jax import lax
from jax.experimental import pallas as pl
from jax.experimental.pallas import tpu as pltpu
from jax.experimental.pallas import tpu_sc as plsc

NC = 2
NS = 16
L = 16
NW = NC * NS
C = 128
CW = 128


def _make_seg_sum(n_rows, d, n_pad, chunks):
    rpt = n_pad // NS
    assert rpt % C == 0
    mesh = plsc.VectorSubcoreMesh(core_axis_name="c", subcore_axis_name="s")

    assert chunks % 2 == 0
    out_type = [jax.ShapeDtypeStruct((NC, n_pad, d), jnp.float32)]
    scratch = [
        pltpu.VMEM((2, C), jnp.int32),
        pltpu.VMEM((2, 1, C), jnp.int32),
        pltpu.VMEM((2, C, d), jnp.float32),
        pltpu.VMEM_SHARED((n_pad, d), jnp.float32),
        pltpu.SemaphoreType.DMA,
        pltpu.SemaphoreType.DMA,
        pltpu.SemaphoreType.DMA,
        pltpu.SemaphoreType.DMA,
        pltpu.SemaphoreType.DMA,
    ]

    def body(table, srcp, dstp, out, src_v, dst_v, rows_v, acc,
             gsem, ssem0, ssem1, dsem0, dsem1):
        cid = lax.axis_index("c")
        sid = lax.axis_index("s")
        wid = sid * NC + cid
        base = sid * rpt
        ssem = (ssem0, ssem1)
        dsem = (dsem0, dsem1)

        def zero_rows(i, _):
            for j in range(d // L):
                rows_v[0, i, pl.ds(j * L, L)] = jnp.zeros((L,), jnp.float32)
            return 0
        lax.fori_loop(0, C, zero_rows, 0)
        for k in range(rpt // C):
            pltpu.sync_copy(rows_v.at[0], acc.at[pl.ds(base + k * C, C)])
        plsc.subcore_barrier()

        pltpu.sync_copy(srcp.at[wid, 0], src_v.at[0])
        pltpu.async_copy(table.at[src_v.at[0]], rows_v.at[0], gsem)
        pltpu.async_copy(srcp.at[wid, 1], src_v.at[1], ssem[1])
        pltpu.async_copy(dstp.at[wid, pl.ds(0, 1)], dst_v.at[0], dsem[0])
        pltpu.async_copy(dstp.at[wid, pl.ds(1, 1)], dst_v.at[1], dsem[1])

        def half(j, p):
            q = 1 - p
            pltpu.make_async_copy(srcp.at[wid, j + 1], src_v.at[q],
                                  ssem[q]).wait()
            pltpu.make_async_copy(table.at[src_v.at[p]], rows_v.at[p],
                                  gsem).wait()
            pltpu.async_copy(table.at[src_v.at[q]], rows_v.at[q], gsem)
            pltpu.async_copy(srcp.at[wid, j + 2], src_v.at[p], ssem[p])
            pltpu.make_async_copy(dstp.at[wid, pl.ds(j, 1)], dst_v.at[p],
                                  dsem[p]).wait()
            pltpu.sync_copy(rows_v.at[p], acc.at[dst_v.at[p].at[0]], add=True)
            pltpu.async_copy(dstp.at[wid, pl.ds(j + 2, 1)], dst_v.at[p],
                             dsem[p])

        def pair(jj, _):
            half(2 * jj, 0)
            half(2 * jj + 1, 1)
            return 0
        lax.fori_loop(0, chunks // 2, pair, 0)
        pltpu.make_async_copy(table.at[src_v.at[0]], rows_v.at[0], gsem).wait()
        pltpu.make_async_copy(srcp.at[wid, 0], src_v.at[1], ssem[1]).wait()
        pltpu.make_async_copy(dstp.at[wid, pl.ds(0, 1)], dst_v.at[0],
                              dsem[0]).wait()
        pltpu.make_async_copy(dstp.at[wid, pl.ds(0, 1)], dst_v.at[1],
                              dsem[1]).wait()
        plsc.subcore_barrier()

        for k in range(rpt // C):
            pltpu.sync_copy(acc.at[pl.ds(base + k * C, C)], rows_v.at[0])
            pltpu.sync_copy(rows_v.at[0], out.at[cid, pl.ds(base + k * C, C)])

    return pl.kernel(body, out_type=out_type, mesh=mesh,
                     scratch_types=scratch)


def _make_counts(n_pad, chunks, w):
    rpt = n_pad // NS
    assert rpt % C == 0
    mesh = plsc.VectorSubcoreMesh(core_axis_name="c", subcore_axis_name="s")

    out_type = [jax.ShapeDtypeStruct((NC, n_pad, w), jnp.float32)]
    scratch = [
        pltpu.VMEM((2, 1, C), jnp.int32),
        pltpu.VMEM((C, w), jnp.float32),
        pltpu.VMEM_SHARED((n_pad, w), jnp.float32),
        pltpu.SemaphoreType.DMA,
        pltpu.SemaphoreType.DMA,
    ]

    def body(dstp, outc, dst_v, ones_v, accc, dsem0, dsem1):
        cid = lax.axis_index("c")
        sid = lax.axis_index("s")
        wid = sid * NC + cid
        base = sid * rpt
        dsem = (dsem0, dsem1)

        def fill(i, _, val):
            for j in range(w // L):
                ones_v[i, pl.ds(j * L, L)] = jnp.full((L,), val, jnp.float32)
            return 0
        lax.fori_loop(0, C, functools.partial(fill, val=0.0), 0)
        for k in range(rpt // C):
            pltpu.sync_copy(ones_v, accc.at[pl.ds(base + k * C, C)])
        lax.fori_loop(0, C, functools.partial(fill, val=1.0), 0)
        plsc.subcore_barrier()

        pltpu.async_copy(dstp.at[wid, pl.ds(0, 1)], dst_v.at[0], dsem[0])
        pltpu.async_copy(dstp.at[wid, pl.ds(1, 1)], dst_v.at[1], dsem[1])

        def half(j, p):
            pltpu.make_async_copy(dstp.at[wid, pl.ds(j, 1)], dst_v.at[p],
                                  dsem[p]).wait()
            pltpu.sync_copy(ones_v, accc.at[dst_v.at[p].at[0]], add=True)
            pltpu.async_copy(dstp.at[wid, pl.ds(j + 2, 1)], dst_v.at[p],
                             dsem[p])

        def pair(jj, _):
            half(2 * jj, 0)
            half(2 * jj + 1, 1)
            return 0
        lax.fori_loop(0, chunks // 2, pair, 0)
        pltpu.make_async_copy(dstp.at[wid, pl.ds(0, 1)], dst_v.at[0],
                              dsem[0]).wait()
        pltpu.make_async_copy(dstp.at[wid, pl.ds(0, 1)], dst_v.at[1],
                              dsem[1]).wait()
        plsc.subcore_barrier()

        for k in range(rpt // C):
            pltpu.sync_copy(accc.at[pl.ds(base + k * C, C)], ones_v)
            pltpu.sync_copy(ones_v, outc.at[cid, pl.ds(base + k * C, C)])

    return pl.kernel(body, out_type=out_type, mesh=mesh,
                     scratch_types=scratch)


def _dot(a, b):
    return jnp.dot(a, b, preferred_element_type=jnp.float32)


def _tc1_body(x_ref, wl_ref, wr_ref, b_ref, y1_ref, xr_ref):
    xb = x_ref[...]
    y1_ref[...] = _dot(xb, wl_ref[...])
    xr_ref[...] = _dot(xb, wr_ref[...]) + b_ref[...]


def _tc2_body(p_ref, pc_ref, xr_ref, wl_ref, wr_ref, b_ref, y2_ref, hr_ref):
    cnt = pc_ref[0, :, 0:1] + pc_ref[1, :, 0:1]
    inv = 1.0 / jnp.maximum(cnt, 1.0)
    h = jnp.maximum((p_ref[0] + p_ref[1]) * inv + xr_ref[...], 0.0)
    y2_ref[...] = _dot(h, wl_ref[...])
    hr_ref[...] = _dot(h, wr_ref[...]) + b_ref[...]


def _tc3_body(q_ref, pc_ref, hr_ref, wlin_ref, blin_ref, out_ref):
    cnt = pc_ref[0, :, 0:1] + pc_ref[1, :, 0:1]
    inv = 1.0 / jnp.maximum(cnt, 1.0)
    h2 = (q_ref[0] + q_ref[1]) * inv + hr_ref[...]
    out_ref[...] = _dot(h2, wlin_ref[...]) + blin_ref[...]


def kernel(x, edge_index, W1l, b1l, W1r, W2l, b2l, W2r, Wlin, blin):
    n, d = x.shape
    e = edge_index.shape[1]
    h_dim = W1l.shape[1]
    o_dim = Wlin.shape[1]

    chunks = -(-e // (NW * C))
    chunks += chunks % 2
    ep = NW * chunks * C
    n_pad = -(-(n + 1) // (NS * C)) * (NS * C)

    src = edge_index[0]
    dst = edge_index[1]
    pad = ep - e
    srcp = jnp.concatenate([src, jnp.zeros((pad,), jnp.int32)]).reshape(
        NW, chunks, C)
    dstp = jnp.concatenate([dst, jnp.full((pad,), n, jnp.int32)]).reshape(
        NW, chunks, C)
    srcp = jnp.concatenate(
        [srcp, jnp.zeros((NW, 2, C), jnp.int32)], axis=1)
    dstp = jnp.concatenate(
        [dstp, jnp.full((NW, 2, C), n, jnp.int32)], axis=1)

    bn = 2000
    grid = (n // bn,)
    row_spec = pl.BlockSpec((bn, h_dim), lambda i: (i, 0))
    w_spec = pl.BlockSpec((d, h_dim), lambda i: (0, 0))
    b_spec = pl.BlockSpec((1, h_dim), lambda i: (0, 0))
    part_spec = pl.BlockSpec((NC, bn, h_dim), lambda i: (0, i, 0))
    cnt_spec = pl.BlockSpec((NC, bn, CW), lambda i: (0, i, 0))

    y1, xr1 = pl.pallas_call(
        _tc1_body,
        grid=grid,
        in_specs=[pl.BlockSpec((bn, d), lambda i: (i, 0)), w_spec, w_spec,
                  b_spec],
        out_specs=[row_spec, row_spec],
        out_shape=[jax.ShapeDtypeStruct((n, h_dim), jnp.float32)] * 2,
    )(x, W1l, W1r, b1l.reshape(1, h_dim))

    (pc,) = _make_counts(n_pad, chunks, CW)(dstp)
    seg1 = _make_seg_sum(n, h_dim, n_pad, chunks)
    (p,) = seg1(y1, srcp, dstp)

    y2, hr2 = pl.pallas_call(
        _tc2_body,
        grid=grid,
        in_specs=[part_spec, cnt_spec, row_spec, w_spec, w_spec, b_spec],
        out_specs=[row_spec, row_spec],
        out_shape=[jax.ShapeDtypeStruct((n, h_dim), jnp.float32)] * 2,
    )(p, pc, xr1, W2l, W2r, b2l.reshape(1, h_dim))

    seg2 = _make_seg_sum(n, h_dim, n_pad, chunks)
    (q,) = seg2(y2, srcp, dstp)

    out = pl.pallas_call(
        _tc3_body,
        grid=grid,
        in_specs=[part_spec, cnt_spec, row_spec,
                  pl.BlockSpec((h_dim, o_dim), lambda i: (0, 0)),
                  pl.BlockSpec((1, o_dim), lambda i: (0, 0))],
        out_specs=pl.BlockSpec((bn, o_dim), lambda i: (i, 0)),
        out_shape=jax.ShapeDtypeStruct((n, o_dim), jnp.float32),
    )(q, pc, hr2, Wlin, blin.reshape(1, o_dim))

    return out


# speedup vs baseline: 1.2662x; 1.2662x over previous
import functools

import jax
import jax.numpy as jnp
from jax import lax
from jax.experimental import pallas as pl
from jax.experimental.pallas import tpu as pltpu
from jax.experimental.pallas import tpu_sc as plsc

NC = 2
NS = 16
L = 16
NW = NC * NS
C = 128
CW = 128
B = 8


def _make_seg_sum(n_rows, d, n_pad, chunks):
    rpt = n_pad // NS
    assert rpt % C == 0
    mesh = plsc.VectorSubcoreMesh(core_axis_name="c", subcore_axis_name="s")

    assert chunks % B == 0
    nblk = chunks // B
    out_type = [jax.ShapeDtypeStruct((NC, n_pad, d), jnp.float32)]
    scratch = [
        pltpu.VMEM((2, B, C), jnp.int32),
        pltpu.VMEM((2, B, 1, C), jnp.int32),
        pltpu.VMEM((2, C, d), jnp.float32),
        pltpu.VMEM_SHARED((n_pad, d), jnp.float32),
        pltpu.SemaphoreType.DMA,
        pltpu.SemaphoreType.DMA,
        pltpu.SemaphoreType.DMA,
        pltpu.SemaphoreType.DMA,
    ]

    def body(table, srcp, dstp, out, src_v, dst_v, rows_v, acc,
             gsem, ssem, isem0, isem1):
        cid = lax.axis_index("c")
        sid = lax.axis_index("s")
        wid = sid * NC + cid
        base = sid * rpt
        isem = (isem0, isem1)

        def zero_rows(i, _):
            for j in range(d // L):
                rows_v[0, i, pl.ds(j * L, L)] = jnp.zeros((L,), jnp.float32)
            return 0
        lax.fori_loop(0, C, zero_rows, 0)
        for k in range(rpt // C):
            pltpu.sync_copy(rows_v.at[0], acc.at[pl.ds(base + k * C, C)])
        plsc.subcore_barrier()

        def idx_fetch(b, s):
            pltpu.async_copy(srcp.at[wid, pl.ds(b * B, B)], src_v.at[s],
                             isem[s])
            pltpu.async_copy(dstp.at[wid, pl.ds(b * B, B)], dst_v.at[s],
                             isem[s])

        def idx_wait(s):
            for _ in range(2):
                pltpu.make_async_copy(srcp.at[wid, pl.ds(0, B)],
                                      src_v.at[s], isem[s]).wait()

        idx_fetch(0, 0)

        def block(b, s):
            idx_fetch(b + 1, 1 - s)
            idx_wait(s)
            for k in range(B):
                pltpu.async_copy(table.at[src_v.at[s, k]], rows_v.at[k % 2],
                                 gsem)
                pltpu.async_copy(rows_v.at[k % 2],
                                 acc.at[dst_v.at[s, k, 0]], ssem, add=True)
            for k in range(B):
                pltpu.make_async_copy(table.at[src_v.at[0, 0]],
                                      rows_v.at[0], gsem).wait()
                pltpu.make_async_copy(rows_v.at[0],
                                      acc.at[dst_v.at[0, 0, 0]], ssem).wait()

        def bpair(bb, _):
            block(2 * bb, 0)
            block(2 * bb + 1, 1)
            return 0
        assert nblk % 2 == 0
        lax.fori_loop(0, nblk // 2, bpair, 0)
        idx_wait(nblk % 2)
        plsc.subcore_barrier()

        for k in range(rpt // C):
            pltpu.sync_copy(acc.at[pl.ds(base + k * C, C)], rows_v.at[0])
            pltpu.sync_copy(rows_v.at[0], out.at[cid, pl.ds(base + k * C, C)])

    return pl.kernel(body, out_type=out_type, mesh=mesh,
                     scratch_types=scratch)


def _make_counts(n_pad, chunks, w):
    rpt = n_pad // NS
    assert rpt % C == 0
    mesh = plsc.VectorSubcoreMesh(core_axis_name="c", subcore_axis_name="s")

    assert chunks % B == 0
    nblk = chunks // B
    assert nblk % 2 == 0
    out_type = [jax.ShapeDtypeStruct((NC, n_pad, w), jnp.float32)]
    scratch = [
        pltpu.VMEM((2, B, 1, C), jnp.int32),
        pltpu.VMEM((C, w), jnp.float32),
        pltpu.VMEM_SHARED((n_pad, w), jnp.float32),
        pltpu.SemaphoreType.DMA,
        pltpu.SemaphoreType.DMA,
        pltpu.SemaphoreType.DMA,
    ]

    def body(dstp, outc, dst_v, ones_v, accc, ssem, isem0, isem1):
        cid = lax.axis_index("c")
        sid = lax.axis_index("s")
        wid = sid * NC + cid
        base = sid * rpt
        isem = (isem0, isem1)

        def fill(i, _, val):
            for j in range(w // L):
                ones_v[i, pl.ds(j * L, L)] = jnp.full((L,), val, jnp.float32)
            return 0
        lax.fori_loop(0, C, functools.partial(fill, val=0.0), 0)
        for k in range(rpt // C):
            pltpu.sync_copy(ones_v, accc.at[pl.ds(base + k * C, C)])
        lax.fori_loop(0, C, functools.partial(fill, val=1.0), 0)
        plsc.subcore_barrier()

        def idx_fetch(b, s):
            pltpu.async_copy(dstp.at[wid, pl.ds(b * B, B)], dst_v.at[s],
                             isem[s])

        def idx_wait(s):
            pltpu.make_async_copy(dstp.at[wid, pl.ds(0, B)], dst_v.at[s],
                                  isem[s]).wait()

        idx_fetch(0, 0)

        def block(b, s):
            idx_fetch(b + 1, 1 - s)
            idx_wait(s)
            for k in range(B):
                pltpu.async_copy(ones_v, accc.at[dst_v.at[s, k, 0]], ssem,
                                 add=True)
            for k in range(B):
                pltpu.make_async_copy(ones_v, accc.at[dst_v.at[0, 0, 0]],
                                      ssem).wait()

        def bpair(bb, _):
            block(2 * bb, 0)
            block(2 * bb + 1, 1)
            return 0
        lax.fori_loop(0, nblk // 2, bpair, 0)
        idx_wait(nblk % 2)
        plsc.subcore_barrier()

        for k in range(rpt // C):
            pltpu.sync_copy(accc.at[pl.ds(base + k * C, C)], ones_v)
            pltpu.sync_copy(ones_v, outc.at[cid, pl.ds(base + k * C, C)])

    return pl.kernel(body, out_type=out_type, mesh=mesh,
                     scratch_types=scratch)


def _dot(a, b):
    return jnp.dot(a, b, preferred_element_type=jnp.float32)


def _tc1_body(x_ref, wl_ref, wr_ref, b_ref, y1_ref, xr_ref):
    xb = x_ref[...]
    y1_ref[...] = _dot(xb, wl_ref[...])
    xr_ref[...] = _dot(xb, wr_ref[...]) + b_ref[...]


def _tc2_body(p_ref, pc_ref, xr_ref, wl_ref, wr_ref, b_ref, y2_ref, hr_ref):
    cnt = pc_ref[0, :, 0:1] + pc_ref[1, :, 0:1]
    inv = 1.0 / jnp.maximum(cnt, 1.0)
    h = jnp.maximum((p_ref[0] + p_ref[1]) * inv + xr_ref[...], 0.0)
    y2_ref[...] = _dot(h, wl_ref[...])
    hr_ref[...] = _dot(h, wr_ref[...]) + b_ref[...]


def _tc3_body(q_ref, pc_ref, hr_ref, wlin_ref, blin_ref, out_ref):
    cnt = pc_ref[0, :, 0:1] + pc_ref[1, :, 0:1]
    inv = 1.0 / jnp.maximum(cnt, 1.0)
    h2 = (q_ref[0] + q_ref[1]) * inv + hr_ref[...]
    out_ref[...] = _dot(h2, wlin_ref[...]) + blin_ref[...]


def kernel(x, edge_index, W1l, b1l, W1r, W2l, b2l, W2r, Wlin, blin):
    n, d = x.shape
    e = edge_index.shape[1]
    h_dim = W1l.shape[1]
    o_dim = Wlin.shape[1]

    chunks = -(-e // (NW * C * 2 * B)) * 2 * B
    ep = NW * chunks * C
    n_pad = -(-(n + 1) // (NS * C)) * (NS * C)

    src = edge_index[0]
    dst = edge_index[1]
    pad = ep - e
    srcp = jnp.concatenate([src, jnp.zeros((pad,), jnp.int32)]).reshape(
        NW, chunks, C)
    dstp = jnp.concatenate([dst, jnp.full((pad,), n, jnp.int32)]).reshape(
        NW, chunks, C)
    srcp = jnp.concatenate(
        [srcp, jnp.zeros((NW, B, C), jnp.int32)], axis=1)
    dstp = jnp.concatenate(
        [dstp, jnp.full((NW, B, C), n, jnp.int32)], axis=1)
    dstp = dstp.reshape(NW, chunks + B, 1, C)

    bn = 2000
    grid = (n // bn,)
    row_spec = pl.BlockSpec((bn, h_dim), lambda i: (i, 0))
    w_spec = pl.BlockSpec((d, h_dim), lambda i: (0, 0))
    b_spec = pl.BlockSpec((1, h_dim), lambda i: (0, 0))
    part_spec = pl.BlockSpec((NC, bn, h_dim), lambda i: (0, i, 0))
    cnt_spec = pl.BlockSpec((NC, bn, CW), lambda i: (0, i, 0))

    y1, xr1 = pl.pallas_call(
        _tc1_body,
        grid=grid,
        in_specs=[pl.BlockSpec((bn, d), lambda i: (i, 0)), w_spec, w_spec,
                  b_spec],
        out_specs=[row_spec, row_spec],
        out_shape=[jax.ShapeDtypeStruct((n, h_dim), jnp.float32)] * 2,
    )(x, W1l, W1r, b1l.reshape(1, h_dim))

    (pc,) = _make_counts(n_pad, chunks, CW)(dstp)
    seg1 = _make_seg_sum(n, h_dim, n_pad, chunks)
    (p,) = seg1(y1, srcp, dstp)

    y2, hr2 = pl.pallas_call(
        _tc2_body,
        grid=grid,
        in_specs=[part_spec, cnt_spec, row_spec, w_spec, w_spec, b_spec],
        out_specs=[row_spec, row_spec],
        out_shape=[jax.ShapeDtypeStruct((n, h_dim), jnp.float32)] * 2,
    )(p, pc, xr1, W2l, W2r, b2l.reshape(1, h_dim))

    seg2 = _make_seg_sum(n, h_dim, n_pad, chunks)
    (q,) = seg2(y2, srcp, dstp)

    out = pl.pallas_call(
        _tc3_body,
        grid=grid,
        in_specs=[part_spec, cnt_spec, row_spec,
                  pl.BlockSpec((h_dim, o_dim), lambda i: (0, 0)),
                  pl.BlockSpec((1, o_dim), lambda i: (0, 0))],
        out_specs=pl.BlockSpec((bn, o_dim), lambda i: (i, 0)),
        out_shape=jax.ShapeDtypeStruct((n, o_dim), jnp.float32),
    )(q, pc, hr2, Wlin, blin.reshape(1, o_dim))

    return out


# speedup vs baseline: 1.3447x; 1.0620x over previous
import functools

import jax
import jax.numpy as jnp
from jax import lax
from jax.experimental import pallas as pl
from jax.experimental.pallas import tpu as pltpu
from jax.experimental.pallas import tpu_sc as plsc

NC = 2
NS = 16
L = 16
NW = NC * NS
C = 128
CW = 128
B = 16


def _make_seg_sum(n_rows, d, n_pad, chunks):
    rpt = n_pad // NS
    assert rpt % C == 0
    mesh = plsc.VectorSubcoreMesh(core_axis_name="c", subcore_axis_name="s")

    assert chunks % B == 0
    nblk = chunks // B
    out_type = [jax.ShapeDtypeStruct((NC, n_pad, d), jnp.float32)]
    scratch = [
        pltpu.VMEM((B, 2, C), jnp.int32),
        pltpu.VMEM((C, d), jnp.float32),
        pltpu.VMEM_SHARED((n_pad, d), jnp.float32),
    ]

    def body(table, edgp, out, edg_v, rows_v, acc):
        cid = lax.axis_index("c")
        sid = lax.axis_index("s")
        wid = sid * NC + cid
        base = sid * rpt

        def zero_rows(i, _):
            for j in range(d // L):
                rows_v[i, pl.ds(j * L, L)] = jnp.zeros((L,), jnp.float32)
            return 0
        lax.fori_loop(0, C, zero_rows, 0)
        for k in range(rpt // C):
            pltpu.sync_copy(rows_v, acc.at[pl.ds(base + k * C, C)])
        plsc.subcore_barrier()

        def block(b, _):
            pltpu.sync_copy(edgp.at[wid, pl.ds(b * B, B)], edg_v)
            for k in range(B):
                pltpu.sync_copy(table.at[edg_v.at[k, 0]], rows_v)
                pltpu.sync_copy(rows_v, acc.at[edg_v.at[k, 1]], add=True)
            return 0
        lax.fori_loop(0, nblk, block, 0)
        plsc.subcore_barrier()

        for k in range(rpt // C):
            pltpu.sync_copy(acc.at[pl.ds(base + k * C, C)], rows_v)
            pltpu.sync_copy(rows_v, out.at[cid, pl.ds(base + k * C, C)])

    return pl.kernel(body, out_type=out_type, mesh=mesh,
                     scratch_types=scratch)


def _make_counts(n_pad, chunks, w):
    rpt = n_pad // NS
    assert rpt % C == 0
    mesh = plsc.VectorSubcoreMesh(core_axis_name="c", subcore_axis_name="s")

    assert chunks % B == 0
    nblk = chunks // B
    out_type = [jax.ShapeDtypeStruct((NC, n_pad, w), jnp.float32)]
    scratch = [
        pltpu.VMEM((B, 2, C), jnp.int32),
        pltpu.VMEM((C, w), jnp.float32),
        pltpu.VMEM_SHARED((n_pad, w), jnp.float32),
    ]

    def body(edgp, outc, edg_v, ones_v, accc):
        cid = lax.axis_index("c")
        sid = lax.axis_index("s")
        wid = sid * NC + cid
        base = sid * rpt

        def fill(i, _, val):
            for j in range(w // L):
                ones_v[i, pl.ds(j * L, L)] = jnp.full((L,), val, jnp.float32)
            return 0
        lax.fori_loop(0, C, functools.partial(fill, val=0.0), 0)
        for k in range(rpt // C):
            pltpu.sync_copy(ones_v, accc.at[pl.ds(base + k * C, C)])
        lax.fori_loop(0, C, functools.partial(fill, val=1.0), 0)
        plsc.subcore_barrier()

        def block(b, _):
            pltpu.sync_copy(edgp.at[wid, pl.ds(b * B, B)], edg_v)
            for k in range(B):
                pltpu.sync_copy(ones_v, accc.at[edg_v.at[k, 1]], add=True)
            return 0
        lax.fori_loop(0, nblk, block, 0)
        plsc.subcore_barrier()

        for k in range(rpt // C):
            pltpu.sync_copy(accc.at[pl.ds(base + k * C, C)], ones_v)
            pltpu.sync_copy(ones_v, outc.at[cid, pl.ds(base + k * C, C)])

    return pl.kernel(body, out_type=out_type, mesh=mesh,
                     scratch_types=scratch)


def _dot(a, b):
    return jnp.dot(a, b, preferred_element_type=jnp.float32)


def _tc1_body(x_ref, wl_ref, wr_ref, b_ref, y1_ref, xr_ref):
    xb = x_ref[...]
    y1_ref[...] = _dot(xb, wl_ref[...])
    xr_ref[...] = _dot(xb, wr_ref[...]) + b_ref[...]


def _tc2_body(p_ref, pc_ref, xr_ref, wl_ref, wr_ref, b_ref, y2_ref, hr_ref):
    cnt = pc_ref[0, :, 0:1] + pc_ref[1, :, 0:1]
    inv = 1.0 / jnp.maximum(cnt, 1.0)
    h = jnp.maximum((p_ref[0] + p_ref[1]) * inv + xr_ref[...], 0.0)
    y2_ref[...] = _dot(h, wl_ref[...])
    hr_ref[...] = _dot(h, wr_ref[...]) + b_ref[...]


def _tc3_body(q_ref, pc_ref, hr_ref, wlin_ref, blin_ref, out_ref):
    cnt = pc_ref[0, :, 0:1] + pc_ref[1, :, 0:1]
    inv = 1.0 / jnp.maximum(cnt, 1.0)
    h2 = (q_ref[0] + q_ref[1]) * inv + hr_ref[...]
    out_ref[...] = _dot(h2, wlin_ref[...]) + blin_ref[...]


def kernel(x, edge_index, W1l, b1l, W1r, W2l, b2l, W2r, Wlin, blin):
    n, d = x.shape
    e = edge_index.shape[1]
    h_dim = W1l.shape[1]
    o_dim = Wlin.shape[1]

    chunks = -(-e // (NW * C * B)) * B
    ep = NW * chunks * C
    n_pad = -(-(n + 1) // (NS * C)) * (NS * C)

    src = edge_index[0]
    dst = edge_index[1]
    pad = ep - e
    srcp = jnp.concatenate([src, jnp.zeros((pad,), jnp.int32)]).reshape(
        NW, chunks, C)
    dstp = jnp.concatenate([dst, jnp.full((pad,), n, jnp.int32)]).reshape(
        NW, chunks, C)
    edgp = jnp.stack([srcp, dstp], axis=2)

    bn = 2000
    grid = (n // bn,)
    row_spec = pl.BlockSpec((bn, h_dim), lambda i: (i, 0))
    w_spec = pl.BlockSpec((d, h_dim), lambda i: (0, 0))
    b_spec = pl.BlockSpec((1, h_dim), lambda i: (0, 0))
    part_spec = pl.BlockSpec((NC, bn, h_dim), lambda i: (0, i, 0))
    cnt_spec = pl.BlockSpec((NC, bn, CW), lambda i: (0, i, 0))

    y1, xr1 = pl.pallas_call(
        _tc1_body,
        grid=grid,
        in_specs=[pl.BlockSpec((bn, d), lambda i: (i, 0)), w_spec, w_spec,
                  b_spec],
        out_specs=[row_spec, row_spec],
        out_shape=[jax.ShapeDtypeStruct((n, h_dim), jnp.float32)] * 2,
    )(x, W1l, W1r, b1l.reshape(1, h_dim))

    (pc,) = _make_counts(n_pad, chunks, CW)(edgp)
    seg1 = _make_seg_sum(n, h_dim, n_pad, chunks)
    (p,) = seg1(y1, edgp)

    y2, hr2 = pl.pallas_call(
        _tc2_body,
        grid=grid,
        in_specs=[part_spec, cnt_spec, row_spec, w_spec, w_spec, b_spec],
        out_specs=[row_spec, row_spec],
        out_shape=[jax.ShapeDtypeStruct((n, h_dim), jnp.float32)] * 2,
    )(p, pc, xr1, W2l, W2r, b2l.reshape(1, h_dim))

    seg2 = _make_seg_sum(n, h_dim, n_pad, chunks)
    (q,) = seg2(y2, edgp)

    out = pl.pallas_call(
        _tc3_body,
        grid=grid,
        in_specs=[part_spec, cnt_spec, row_spec,
                  pl.BlockSpec((h_dim, o_dim), lambda i: (0, 0)),
                  pl.BlockSpec((1, o_dim), lambda i: (0, 0))],
        out_specs=pl.BlockSpec((bn, o_dim), lambda i: (i, 0)),
        out_shape=jax.ShapeDtypeStruct((n, o_dim), jnp.float32),
    )(q, pc, hr2, Wlin, blin.reshape(1, o_dim))

    return out


# speedup vs baseline: 1.8062x; 1.3432x over previous
import functools

import jax
import jax.numpy as jnp
from jax import lax
from jax.experimental import pallas as pl
from jax.experimental.pallas import tpu as pltpu
from jax.experimental.pallas import tpu_sc as plsc

NC = 2
NS = 16
L = 16
NW = NC * NS
C = 128
CW = 128
B = 16


def _make_seg_sum(n_rows, d, n_pad, chunks):
    rpt = n_pad // NS
    assert rpt % C == 0
    mesh = plsc.VectorSubcoreMesh(core_axis_name="c", subcore_axis_name="s")

    out_type = [jax.ShapeDtypeStruct((NC, n_pad, d), jnp.float32)]
    scratch = [
        pltpu.VMEM((C,), jnp.int32),
        pltpu.VMEM((1, C), jnp.int32),
        pltpu.VMEM((C, d), jnp.float32),
        pltpu.VMEM_SHARED((n_pad, d), jnp.float32),
    ]

    def body(table, srcp, dstp, out, src_v, dst_v, rows_v, acc):
        cid = lax.axis_index("c")
        sid = lax.axis_index("s")
        wid = sid * NC + cid
        base = sid * rpt

        def zero_rows(i, _):
            for j in range(d // L):
                rows_v[i, pl.ds(j * L, L)] = jnp.zeros((L,), jnp.float32)
            return 0
        lax.fori_loop(0, C, zero_rows, 0)
        for k in range(rpt // C):
            pltpu.sync_copy(rows_v, acc.at[pl.ds(base + k * C, C)])
        plsc.subcore_barrier()

        def step(j, _):
            pltpu.sync_copy(srcp.at[wid, j], src_v)
            pltpu.sync_copy(dstp.at[wid, pl.ds(j, 1)], dst_v)
            pltpu.sync_copy(table.at[src_v], rows_v)
            pltpu.sync_copy(rows_v, acc.at[dst_v.at[0]], add=True)
            return 0
        lax.fori_loop(0, chunks, step, 0)
        plsc.subcore_barrier()

        for k in range(rpt // C):
            pltpu.sync_copy(acc.at[pl.ds(base + k * C, C)], rows_v)
            pltpu.sync_copy(rows_v, out.at[cid, pl.ds(base + k * C, C)])

    return pl.kernel(body, out_type=out_type, mesh=mesh,
                     scratch_types=scratch)


def _make_counts(n_pad, chunks, w):
    rpt = n_pad // NS
    assert rpt % C == 0
    mesh = plsc.VectorSubcoreMesh(core_axis_name="c", subcore_axis_name="s")

    out_type = [jax.ShapeDtypeStruct((NC, n_pad, w), jnp.float32)]
    scratch = [
        pltpu.VMEM((1, C), jnp.int32),
        pltpu.VMEM((C, w), jnp.float32),
        pltpu.VMEM_SHARED((n_pad, w), jnp.float32),
    ]

    def body(dstp, outc, dst_v, ones_v, accc):
        cid = lax.axis_index("c")
        sid = lax.axis_index("s")
        wid = sid * NC + cid
        base = sid * rpt

        def fill(i, _, val):
            for j in range(w // L):
                ones_v[i, pl.ds(j * L, L)] = jnp.full((L,), val, jnp.float32)
            return 0
        lax.fori_loop(0, C, functools.partial(fill, val=0.0), 0)
        for k in range(rpt // C):
            pltpu.sync_copy(ones_v, accc.at[pl.ds(base + k * C, C)])
        lax.fori_loop(0, C, functools.partial(fill, val=1.0), 0)
        plsc.subcore_barrier()

        def step(j, _):
            pltpu.sync_copy(dstp.at[wid, pl.ds(j, 1)], dst_v)
            pltpu.sync_copy(ones_v, accc.at[dst_v.at[0]], add=True)
            return 0
        lax.fori_loop(0, chunks, step, 0)
        plsc.subcore_barrier()

        for k in range(rpt // C):
            pltpu.sync_copy(accc.at[pl.ds(base + k * C, C)], ones_v)
            pltpu.sync_copy(ones_v, outc.at[cid, pl.ds(base + k * C, C)])

    return pl.kernel(body, out_type=out_type, mesh=mesh,
                     scratch_types=scratch)


def _dot(a, b):
    return jnp.dot(a, b, preferred_element_type=jnp.float32)


def _tc1_body(x_ref, wl_ref, wr_ref, b_ref, y1_ref, xr_ref):
    xb = x_ref[...]
    y1_ref[...] = _dot(xb, wl_ref[...])
    xr_ref[...] = _dot(xb, wr_ref[...]) + b_ref[...]


def _tc2_body(p_ref, pc_ref, xr_ref, wl_ref, wr_ref, b_ref, y2_ref, hr_ref):
    cnt = pc_ref[0, :, 0:1] + pc_ref[1, :, 0:1]
    inv = 1.0 / jnp.maximum(cnt, 1.0)
    h = jnp.maximum((p_ref[0] + p_ref[1]) * inv + xr_ref[...], 0.0)
    y2_ref[...] = _dot(h, wl_ref[...])
    hr_ref[...] = _dot(h, wr_ref[...]) + b_ref[...]


def _tc3_body(q_ref, pc_ref, hr_ref, wlin_ref, blin_ref, out_ref):
    cnt = pc_ref[0, :, 0:1] + pc_ref[1, :, 0:1]
    inv = 1.0 / jnp.maximum(cnt, 1.0)
    h2 = (q_ref[0] + q_ref[1]) * inv + hr_ref[...]
    out_ref[...] = _dot(h2, wlin_ref[...]) + blin_ref[...]


def kernel(x, edge_index, W1l, b1l, W1r, W2l, b2l, W2r, Wlin, blin):
    n, d = x.shape
    e = edge_index.shape[1]
    h_dim = W1l.shape[1]
    o_dim = Wlin.shape[1]

    chunks = -(-e // (NW * C))
    ep = NW * chunks * C
    n_pad = -(-(n + 1) // (NS * C)) * (NS * C)

    src = edge_index[0]
    dst = edge_index[1]
    pad = ep - e
    srcp = jnp.concatenate([src, jnp.zeros((pad,), jnp.int32)]).reshape(
        NW, chunks, C)
    dstp = jnp.concatenate([dst, jnp.full((pad,), n, jnp.int32)]).reshape(
        NW, chunks, C)


    bn = 2000
    grid = (n // bn,)
    row_spec = pl.BlockSpec((bn, h_dim), lambda i: (i, 0))
    w_spec = pl.BlockSpec((d, h_dim), lambda i: (0, 0))
    b_spec = pl.BlockSpec((1, h_dim), lambda i: (0, 0))
    part_spec = pl.BlockSpec((NC, bn, h_dim), lambda i: (0, i, 0))
    cnt_spec = pl.BlockSpec((NC, bn, CW), lambda i: (0, i, 0))

    y1, xr1 = pl.pallas_call(
        _tc1_body,
        grid=grid,
        in_specs=[pl.BlockSpec((bn, d), lambda i: (i, 0)), w_spec, w_spec,
                  b_spec],
        out_specs=[row_spec, row_spec],
        out_shape=[jax.ShapeDtypeStruct((n, h_dim), jnp.float32)] * 2,
    )(x, W1l, W1r, b1l.reshape(1, h_dim))

    (pc,) = _make_counts(n_pad, chunks, CW)(dstp)
    seg1 = _make_seg_sum(n, h_dim, n_pad, chunks)
    (p,) = seg1(y1, srcp, dstp)

    y2, hr2 = pl.pallas_call(
        _tc2_body,
        grid=grid,
        in_specs=[part_spec, cnt_spec, row_spec, w_spec, w_spec, b_spec],
        out_specs=[row_spec, row_spec],
        out_shape=[jax.ShapeDtypeStruct((n, h_dim), jnp.float32)] * 2,
    )(p, pc, xr1, W2l, W2r, b2l.reshape(1, h_dim))

    seg2 = _make_seg_sum(n, h_dim, n_pad, chunks)
    (q,) = seg2(y2, srcp, dstp)

    out = pl.pallas_call(
        _tc3_body,
        grid=grid,
        in_specs=[part_spec, cnt_spec, row_spec,
                  pl.BlockSpec((h_dim, o_dim), lambda i: (0, 0)),
                  pl.BlockSpec((1, o_dim), lambda i: (0, 0))],
        out_specs=pl.BlockSpec((bn, o_dim), lambda i: (i, 0)),
        out_shape=jax.ShapeDtypeStruct((n, o_dim), jnp.float32),
    )(q, pc, hr2, Wlin, blin.reshape(1, o_dim))

    return out


# speedup vs baseline: 1.9705x; 1.0910x over previous
import functools

import jax
import jax.numpy as jnp
from jax import lax
from jax.experimental import pallas as pl
from jax.experimental.pallas import tpu as pltpu
from jax.experimental.pallas import tpu_sc as plsc

NC = 2
NS = 16
L = 16
NW = NC * NS
C = 128
CW = 64
B = 16


def _make_seg_sum(n_rows, d, n_pad, chunks):
    rpt = n_pad // NS
    assert rpt % C == 0
    mesh = plsc.VectorSubcoreMesh(core_axis_name="c", subcore_axis_name="s")

    out_type = [jax.ShapeDtypeStruct((NC, n_pad, d), jnp.float32)]
    scratch = [
        pltpu.VMEM((1, 2, C), jnp.int32),
        pltpu.VMEM((C, d), jnp.float32),
        pltpu.VMEM_SHARED((n_pad, d), jnp.float32),
    ]

    def body(table, edgp, out, edg_v, rows_v, acc):
        cid = lax.axis_index("c")
        sid = lax.axis_index("s")
        wid = sid * NC + cid
        base = sid * rpt

        def zero_rows(i, _):
            for j in range(d // L):
                rows_v[i, pl.ds(j * L, L)] = jnp.zeros((L,), jnp.float32)
            return 0
        lax.fori_loop(0, C, zero_rows, 0)
        for k in range(rpt // C):
            pltpu.sync_copy(rows_v, acc.at[pl.ds(base + k * C, C)])
        plsc.subcore_barrier()

        def step(j, _):
            pltpu.sync_copy(edgp.at[wid, pl.ds(j, 1)], edg_v)
            pltpu.sync_copy(table.at[edg_v.at[0, 0]], rows_v)
            pltpu.sync_copy(rows_v, acc.at[edg_v.at[0, 1]], add=True)
            return 0
        lax.fori_loop(0, chunks, step, 0)
        plsc.subcore_barrier()

        for k in range(rpt // C):
            pltpu.sync_copy(acc.at[pl.ds(base + k * C, C)], rows_v)
            pltpu.sync_copy(rows_v, out.at[cid, pl.ds(base + k * C, C)])

    return pl.kernel(body, out_type=out_type, mesh=mesh,
                     scratch_types=scratch)


def _make_counts(n_pad, chunks, w):
    rpt = n_pad // NS
    assert rpt % C == 0
    mesh = plsc.VectorSubcoreMesh(core_axis_name="c", subcore_axis_name="s")

    out_type = [jax.ShapeDtypeStruct((NC, n_pad, w), jnp.float32)]
    scratch = [
        pltpu.VMEM((1, C), jnp.int32),
        pltpu.VMEM((C, w), jnp.float32),
        pltpu.VMEM_SHARED((n_pad, w), jnp.float32),
    ]

    def body(dstp, outc, dst_v, ones_v, accc):
        cid = lax.axis_index("c")
        sid = lax.axis_index("s")
        wid = sid * NC + cid
        base = sid * rpt

        def fill(i, _, val):
            for j in range(w // L):
                ones_v[i, pl.ds(j * L, L)] = jnp.full((L,), val, jnp.float32)
            return 0
        lax.fori_loop(0, C, functools.partial(fill, val=0.0), 0)
        for k in range(rpt // C):
            pltpu.sync_copy(ones_v, accc.at[pl.ds(base + k * C, C)])
        lax.fori_loop(0, C, functools.partial(fill, val=1.0), 0)
        plsc.subcore_barrier()

        def step(j, _):
            pltpu.sync_copy(dstp.at[wid, pl.ds(j, 1)], dst_v)
            pltpu.sync_copy(ones_v, accc.at[dst_v.at[0]], add=True)
            return 0
        lax.fori_loop(0, chunks, step, 0)
        plsc.subcore_barrier()

        for k in range(rpt // C):
            pltpu.sync_copy(accc.at[pl.ds(base + k * C, C)], ones_v)
            pltpu.sync_copy(ones_v, outc.at[cid, pl.ds(base + k * C, C)])

    return pl.kernel(body, out_type=out_type, mesh=mesh,
                     scratch_types=scratch)


def _dot(a, b):
    return jnp.dot(a, b, preferred_element_type=jnp.float32)


def _tc1_body(x_ref, wl_ref, wr_ref, b_ref, y1_ref, xr_ref):
    xb = x_ref[...]
    y1_ref[...] = _dot(xb, wl_ref[...])
    xr_ref[...] = _dot(xb, wr_ref[...]) + b_ref[...]


def _tc2_body(p_ref, pc_ref, xr_ref, wl_ref, wr_ref, b_ref, y2_ref, hr_ref):
    cnt = pc_ref[0, :, 0:1] + pc_ref[1, :, 0:1]
    inv = 1.0 / jnp.maximum(cnt, 1.0)
    h = jnp.maximum((p_ref[0] + p_ref[1]) * inv + xr_ref[...], 0.0)
    y2_ref[...] = _dot(h, wl_ref[...])
    hr_ref[...] = _dot(h, wr_ref[...]) + b_ref[...]


def _tc3_body(q_ref, pc_ref, hr_ref, wlin_ref, blin_ref, out_ref):
    cnt = pc_ref[0, :, 0:1] + pc_ref[1, :, 0:1]
    inv = 1.0 / jnp.maximum(cnt, 1.0)
    h2 = (q_ref[0] + q_ref[1]) * inv + hr_ref[...]
    out_ref[...] = _dot(h2, wlin_ref[...]) + blin_ref[...]


def kernel(x, edge_index, W1l, b1l, W1r, W2l, b2l, W2r, Wlin, blin):
    n, d = x.shape
    e = edge_index.shape[1]
    h_dim = W1l.shape[1]
    o_dim = Wlin.shape[1]

    chunks = -(-e // (NW * C))
    ep = NW * chunks * C
    n_pad = -(-(n + 1) // (NS * C)) * (NS * C)

    src = edge_index[0]
    dst = edge_index[1]
    pad = ep - e
    srcp = jnp.concatenate([src, jnp.zeros((pad,), jnp.int32)]).reshape(
        NW, chunks, C)
    dstp = jnp.concatenate([dst, jnp.full((pad,), n, jnp.int32)]).reshape(
        NW, chunks, C)
    edgp = jnp.stack([srcp, dstp], axis=2)


    bn = 2000
    grid = (n // bn,)
    row_spec = pl.BlockSpec((bn, h_dim), lambda i: (i, 0))
    w_spec = pl.BlockSpec((d, h_dim), lambda i: (0, 0))
    b_spec = pl.BlockSpec((1, h_dim), lambda i: (0, 0))
    part_spec = pl.BlockSpec((NC, bn, h_dim), lambda i: (0, i, 0))
    cnt_spec = pl.BlockSpec((NC, bn, CW), lambda i: (0, i, 0))

    y1, xr1 = pl.pallas_call(
        _tc1_body,
        grid=grid,
        in_specs=[pl.BlockSpec((bn, d), lambda i: (i, 0)), w_spec, w_spec,
                  b_spec],
        out_specs=[row_spec, row_spec],
        out_shape=[jax.ShapeDtypeStruct((n, h_dim), jnp.float32)] * 2,
    )(x, W1l, W1r, b1l.reshape(1, h_dim))

    (pc,) = _make_counts(n_pad, chunks, CW)(dstp)
    seg1 = _make_seg_sum(n, h_dim, n_pad, chunks)
    (p,) = seg1(y1, edgp)

    y2, hr2 = pl.pallas_call(
        _tc2_body,
        grid=grid,
        in_specs=[part_spec, cnt_spec, row_spec, w_spec, w_spec, b_spec],
        out_specs=[row_spec, row_spec],
        out_shape=[jax.ShapeDtypeStruct((n, h_dim), jnp.float32)] * 2,
    )(p, pc, xr1, W2l, W2r, b2l.reshape(1, h_dim))

    seg2 = _make_seg_sum(n, h_dim, n_pad, chunks)
    (q,) = seg2(y2, edgp)

    out = pl.pallas_call(
        _tc3_body,
        grid=grid,
        in_specs=[part_spec, cnt_spec, row_spec,
                  pl.BlockSpec((h_dim, o_dim), lambda i: (0, 0)),
                  pl.BlockSpec((1, o_dim), lambda i: (0, 0))],
        out_specs=pl.BlockSpec((bn, o_dim), lambda i: (i, 0)),
        out_shape=jax.ShapeDtypeStruct((n, o_dim), jnp.float32),
    )(q, pc, hr2, Wlin, blin.reshape(1, o_dim))

    return out
